# baseline (speedup 1.0000x reference)
import functools

import jax
import jax.numpy as jnp
from jax import lax
from jax.experimental import pallas as pl
from jax.experimental.pallas import tpu as pltpu
from jax.experimental.pallas import tpu_sc as plsc
from jax.experimental.layout import Layout, with_layout_constraint

C_LEN = 1_000_000
U_LEN = 100_000
EMBED = 64
BATCH = 16384

C_TILES = -(-C_LEN // 128)
U_TILES = -(-U_LEN // 128)
C_SLAB = C_TILES * 1024
U_SLAB = U_TILES * 1024
C_TSTART = ((EMBED * C_LEN - 7 * C_SLAB - 1024) // 1024 + 1) * 128
U_TSTART = ((EMBED * U_LEN - 7 * U_SLAB - 1024) // 1024 + 1) * 128
DTAIL = 56
C_TLEN = C_LEN - C_TSTART
U_TLEN = U_LEN - U_TSTART
C_TP = -(-C_TLEN // 128) * 128
U_TP = -(-U_TLEN // 128) * 128
CT_SLAB = (C_TP // 128) * 1024
UT_SLAB = (U_TP // 128) * 1024

NUM_CORES = 2
NUM_SUBCORES = 16
NW = NUM_CORES * NUM_SUBCORES
BPW = BATCH // NW
CHUNK = 128
NCH = BPW // CHUNK
LANES = 16
VPC = CHUNK // LANES

_mesh = plsc.VectorSubcoreMesh(core_axis_name="c", subcore_axis_name="s")


@functools.partial(
    pl.kernel,
    mesh=_mesh,
    out_type=jax.ShapeDtypeStruct((BATCH,), jnp.float32),
    compiler_params=pltpu.CompilerParams(
        needs_layout_passes=False, use_tc_tiling_on_sc=False),
    scratch_types=[
        pltpu.VMEM((BPW,), jnp.int32),
        pltpu.VMEM((BPW,), jnp.int32),
        [pltpu.VMEM((EMBED, CHUNK), jnp.int32) for _ in range(2)],
        [pltpu.VMEM((EMBED, CHUNK), jnp.int32) for _ in range(2)],
        [pltpu.VMEM((EMBED - DTAIL, CHUNK), jnp.int32) for _ in range(2)],
        [pltpu.VMEM((EMBED - DTAIL, CHUNK), jnp.int32) for _ in range(2)],
        [pltpu.VMEM((EMBED, CHUNK), jnp.float32) for _ in range(2)],
        [pltpu.VMEM((EMBED, CHUNK), jnp.float32) for _ in range(2)],
        [pltpu.VMEM((EMBED - DTAIL, CHUNK), jnp.float32) for _ in range(2)],
        [pltpu.VMEM((EMBED - DTAIL, CHUNK), jnp.float32) for _ in range(2)],
        pltpu.VMEM((BPW,), jnp.float32),
        pltpu.SemaphoreType.DMA,
    ],
)
def _gmf_sc(c_idx_hbm, u_idx_hbm, c_tab_hbm, u_tab_hbm, ctail_hbm, utail_hbm,
            out_hbm, cidx_v, uidx_v, cib, uib, ctb, utb, cdst, udst,
            ctdst, utdst, out_v, sem):
    wid = lax.axis_index("s") * NUM_CORES + lax.axis_index("c")
    base = wid * BPW

    pltpu.sync_copy(c_idx_hbm.at[pl.ds(base, BPW)], cidx_v)
    pltpu.sync_copy(u_idx_hbm.at[pl.ds(base, BPW)], uidx_v)

    def fire(ch, b):
        for v in range(VPC):
            s = pl.ds(v * LANES, LANES)
            rc = cidx_v[pl.ds(ch * CHUNK + v * LANES, LANES)]
            ru = uidx_v[pl.ds(ch * CHUNK + v * LANES, LANES)]
            tc = ((rc >> 7) << 10) + (rc & 127)
            tu = ((ru >> 7) << 10) + (ru & 127)
            for d in range(EMBED):
                cib[b][d, s] = tc + ((d >> 3) * C_SLAB + (d & 7) * 128)
                uib[b][d, s] = tu + ((d >> 3) * U_SLAB + (d & 7) * 128)
            spread = v * LANES + lax.iota(jnp.int32, LANES)
            rtc = jnp.where(rc >= C_TSTART, rc - C_TSTART, spread)
            rtu = jnp.where(ru >= U_TSTART, ru - U_TSTART, spread)
            for d in range(DTAIL, EMBED):
                ctb[b][d - DTAIL, s] = rtc + d * C_TP
                utb[b][d - DTAIL, s] = rtu + d * U_TP
        copies = []
        for d in range(EMBED):
            copies.append(pltpu.async_copy(
                c_tab_hbm.at[0].at[cib[b].at[d]], cdst[b].at[d], sem))
            copies.append(pltpu.async_copy(
                u_tab_hbm.at[0].at[uib[b].at[d]], udst[b].at[d], sem))
        for j in range(EMBED - DTAIL):
            copies.append(pltpu.async_copy(
                ctail_hbm.at[ctb[b].at[j]], ctdst[b].at[j], sem))
            copies.append(pltpu.async_copy(
                utail_hbm.at[utb[b].at[j]], utdst[b].at[j], sem))
        return copies

    def drain(copies):
        for cp in copies:
            cp.wait()

    def compute(ch, b):
        for v in range(VPC):
            s = pl.ds(v * LANES, LANES)
            rc = cidx_v[pl.ds(ch * CHUNK + v * LANES, LANES)]
            ru = uidx_v[pl.ds(ch * CHUNK + v * LANES, LANES)]
            mc = rc >= C_TSTART
            mu = ru >= U_TSTART
            accs = [None] * 4
            for d in range(EMBED):
                cv = cdst[b][d, s]
                uv = udst[b][d, s]
                if d >= DTAIL:
                    cv = jnp.where(mc, ctdst[b][d - DTAIL, s], cv)
                    uv = jnp.where(mu, utdst[b][d - DTAIL, s], uv)
                p = cv * uv
                k = d % 4
                accs[k] = p if accs[k] is None else accs[k] + p
            out_v[pl.ds(ch * CHUNK + v * LANES, LANES)] = (
                (accs[0] + accs[1]) + (accs[2] + accs[3]))

    def pair_body(i, carry):
        ch = i * 2
        ca = fire(ch, 0)
        drain(ca)
        cb = fire(ch + 1, 1)
        compute(ch, 0)
        drain(cb)
        compute(ch + 1, 1)
        return carry

    lax.fori_loop(0, NCH // 2, pair_body, 0)

    pltpu.sync_copy(out_v, out_hbm.at[pl.ds(base, BPW)])


def kernel(c_idx, u_idx, c_table, u_table):
    c_idx32 = jnp.asarray(c_idx, jnp.int32)
    u_idx32 = jnp.asarray(u_idx, jnp.int32)
    fmt = Layout(major_to_minor=(0, 1), tiling=((8, 128),))
    ct = with_layout_constraint(c_table.T, fmt)
    ut = with_layout_constraint(u_table.T, fmt)
    ctail = jnp.ravel(
        jnp.pad(c_table.T[:, C_TSTART:], ((0, 0), (0, C_TP - C_TLEN))))
    utail = jnp.ravel(
        jnp.pad(u_table.T[:, U_TSTART:], ((0, 0), (0, U_TP - U_TLEN))))
    out = _gmf_sc(c_idx32, u_idx32, ct, ut, ctail, utail)
    return out.reshape(BATCH, 1)


# speedup vs baseline: 1.0650x; 1.0650x over previous
import functools

import jax
import jax.numpy as jnp
from jax import lax
from jax.experimental import pallas as pl
from jax.experimental.pallas import tpu as pltpu
from jax.experimental.pallas import tpu_sc as plsc
from jax.experimental.layout import Layout, with_layout_constraint

C_LEN = 1_000_000
U_LEN = 100_000
EMBED = 64
BATCH = 16384

C_TILES = -(-C_LEN // 128)
U_TILES = -(-U_LEN // 128)
C_SLAB = C_TILES * 1024
U_SLAB = U_TILES * 1024
C_TSTART = ((EMBED * C_LEN - 7 * C_SLAB - 1024) // 1024 + 1) * 128
U_TSTART = ((EMBED * U_LEN - 7 * U_SLAB - 1024) // 1024 + 1) * 128
DTAIL = 56
C_TLEN = C_LEN - C_TSTART
U_TLEN = U_LEN - U_TSTART
C_TP = -(-C_TLEN // 128) * 128
U_TP = -(-U_TLEN // 128) * 128
CT_SLAB = (C_TP // 128) * 1024
UT_SLAB = (U_TP // 128) * 1024

NUM_CORES = 2
NUM_SUBCORES = 16
NW = NUM_CORES * NUM_SUBCORES
BPW = BATCH // NW
CHUNK = 128
NCH = BPW // CHUNK
LANES = 16
VPC = CHUNK // LANES

_mesh = plsc.VectorSubcoreMesh(core_axis_name="c", subcore_axis_name="s")


@functools.partial(
    pl.kernel,
    mesh=_mesh,
    out_type=jax.ShapeDtypeStruct((BATCH,), jnp.float32),
    compiler_params=pltpu.CompilerParams(
        needs_layout_passes=False, use_tc_tiling_on_sc=False),
    scratch_types=[
        pltpu.VMEM((BPW,), jnp.int32),
        pltpu.VMEM((BPW,), jnp.int32),
        [pltpu.VMEM((EMBED, CHUNK), jnp.int32) for _ in range(2)],
        [pltpu.VMEM((EMBED, CHUNK), jnp.int32) for _ in range(2)],
        [pltpu.VMEM((EMBED - DTAIL, CHUNK), jnp.int32) for _ in range(2)],
        [pltpu.VMEM((EMBED - DTAIL, CHUNK), jnp.int32) for _ in range(2)],
        [pltpu.VMEM((EMBED, CHUNK), jnp.float32) for _ in range(2)],
        [pltpu.VMEM((EMBED, CHUNK), jnp.float32) for _ in range(2)],
        [pltpu.VMEM((EMBED - DTAIL, CHUNK), jnp.float32) for _ in range(2)],
        [pltpu.VMEM((EMBED - DTAIL, CHUNK), jnp.float32) for _ in range(2)],
        pltpu.VMEM((BPW,), jnp.float32),
        pltpu.SemaphoreType.DMA,
    ],
)
def _gmf_sc(c_idx_hbm, u_idx_hbm, c_tab_hbm, u_tab_hbm, ctail_hbm, utail_hbm,
            out_hbm, cidx_v, uidx_v, cib, uib, ctb, utb, cdst, udst,
            ctdst, utdst, out_v, sem):
    wid = lax.axis_index("s") * NUM_CORES + lax.axis_index("c")
    base = wid * BPW

    pltpu.sync_copy(c_idx_hbm.at[pl.ds(base, BPW)], cidx_v)
    pltpu.sync_copy(u_idx_hbm.at[pl.ds(base, BPW)], uidx_v)

    def fire(ch, b):
        for v in range(VPC):
            s = pl.ds(v * LANES, LANES)
            rc = cidx_v[pl.ds(ch * CHUNK + v * LANES, LANES)]
            ru = uidx_v[pl.ds(ch * CHUNK + v * LANES, LANES)]
            tc = ((rc >> 7) << 10) + (rc & 127)
            tu = ((ru >> 7) << 10) + (ru & 127)
            for d in range(EMBED):
                cib[b][d, s] = tc + ((d >> 3) * C_SLAB + (d & 7) * 128)
                uib[b][d, s] = tu + ((d >> 3) * U_SLAB + (d & 7) * 128)
            spread = v * LANES + lax.iota(jnp.int32, LANES)
            rtc = jnp.where(rc >= C_TSTART, rc - C_TSTART, spread)
            rtu = jnp.where(ru >= U_TSTART, ru - U_TSTART, spread)
            rtc8 = rtc << 3
            rtu8 = rtu << 3
            for d in range(DTAIL, EMBED):
                ctb[b][d - DTAIL, s] = rtc8 + (d - DTAIL)
                utb[b][d - DTAIL, s] = rtu8 + (d - DTAIL)
        copies = []
        for d in range(EMBED):
            copies.append(pltpu.async_copy(
                c_tab_hbm.at[0].at[cib[b].at[d]], cdst[b].at[d], sem))
            copies.append(pltpu.async_copy(
                u_tab_hbm.at[0].at[uib[b].at[d]], udst[b].at[d], sem))
        for j in range(EMBED - DTAIL):
            copies.append(pltpu.async_copy(
                ctail_hbm.at[ctb[b].at[j]], ctdst[b].at[j], sem))
            copies.append(pltpu.async_copy(
                utail_hbm.at[utb[b].at[j]], utdst[b].at[j], sem))
        return copies

    def drain(copies):
        for cp in copies:
            cp.wait()

    def compute(ch, b):
        for v in range(VPC):
            s = pl.ds(v * LANES, LANES)
            rc = cidx_v[pl.ds(ch * CHUNK + v * LANES, LANES)]
            ru = uidx_v[pl.ds(ch * CHUNK + v * LANES, LANES)]
            mc = rc >= C_TSTART
            mu = ru >= U_TSTART
            accs = [None] * 4
            for d in range(EMBED):
                cv = cdst[b][d, s]
                uv = udst[b][d, s]
                if d >= DTAIL:
                    cv = jnp.where(mc, ctdst[b][d - DTAIL, s], cv)
                    uv = jnp.where(mu, utdst[b][d - DTAIL, s], uv)
                p = cv * uv
                k = d % 4
                accs[k] = p if accs[k] is None else accs[k] + p
            out_v[pl.ds(ch * CHUNK + v * LANES, LANES)] = (
                (accs[0] + accs[1]) + (accs[2] + accs[3]))

    def pair_body(i, carry):
        ch = i * 2
        ca = fire(ch, 0)
        drain(ca)
        cb = fire(ch + 1, 1)
        compute(ch, 0)
        drain(cb)
        compute(ch + 1, 1)
        return carry

    lax.fori_loop(0, NCH // 2, pair_body, 0)

    pltpu.sync_copy(out_v, out_hbm.at[pl.ds(base, BPW)])


def kernel(c_idx, u_idx, c_table, u_table):
    c_idx32 = jnp.asarray(c_idx, jnp.int32)
    u_idx32 = jnp.asarray(u_idx, jnp.int32)
    fmt = Layout(major_to_minor=(0, 1), tiling=((8, 128),))
    ct = with_layout_constraint(c_table.T, fmt)
    ut = with_layout_constraint(u_table.T, fmt)
    ctail = jnp.ravel(jnp.pad(
        c_table.T[DTAIL:, C_TSTART:], ((0, 0), (0, C_TP - C_TLEN))).T)
    utail = jnp.ravel(jnp.pad(
        u_table.T[DTAIL:, U_TSTART:], ((0, 0), (0, U_TP - U_TLEN))).T)
    out = _gmf_sc(c_idx32, u_idx32, ct, ut, ctail, utail)
    return out.reshape(BATCH, 1)


# speedup vs baseline: 1.2281x; 1.1531x over previous
import functools

import jax
import jax.numpy as jnp
from jax import lax
from jax.experimental import pallas as pl
from jax.experimental.pallas import tpu as pltpu
from jax.experimental.pallas import tpu_sc as plsc
from jax.experimental.layout import Layout, with_layout_constraint

C_LEN = 1_000_000
U_LEN = 100_000
EMBED = 64
BATCH = 16384

C_TILES = -(-C_LEN // 128)
U_TILES = -(-U_LEN // 128)
C_SLAB = C_TILES * 1024
U_SLAB = U_TILES * 1024
C_TSTART = ((EMBED * C_LEN - 7 * C_SLAB - 1024) // 1024 + 1) * 128
U_TSTART = ((EMBED * U_LEN - 7 * U_SLAB - 1024) // 1024 + 1) * 128
DTAIL = 56
C_TLEN = C_LEN - C_TSTART
U_TLEN = U_LEN - U_TSTART
C_TP = -(-C_TLEN // 128) * 128
U_TP = -(-U_TLEN // 128) * 128
CT_SLAB = (C_TP // 128) * 1024
UT_SLAB = (U_TP // 128) * 1024

NUM_CORES = 2
NUM_SUBCORES = 16
NW = NUM_CORES * NUM_SUBCORES
BPW = BATCH // NW
CHUNK = 128
NCH = BPW // CHUNK
LANES = 16
VPC = CHUNK // LANES

_mesh = plsc.VectorSubcoreMesh(core_axis_name="c", subcore_axis_name="s")


@functools.partial(
    pl.kernel,
    mesh=_mesh,
    out_type=jax.ShapeDtypeStruct((BATCH,), jnp.float32),
    compiler_params=pltpu.CompilerParams(
        needs_layout_passes=False, use_tc_tiling_on_sc=False),
    scratch_types=[
        pltpu.VMEM((BPW,), jnp.int32),
        pltpu.VMEM((BPW,), jnp.int32),
        [pltpu.VMEM((EMBED, CHUNK), jnp.int32) for _ in range(2)],
        [pltpu.VMEM((EMBED, CHUNK), jnp.int32) for _ in range(2)],
        [pltpu.VMEM((EMBED - DTAIL, CHUNK), jnp.int32) for _ in range(2)],
        [pltpu.VMEM((EMBED - DTAIL, CHUNK), jnp.int32) for _ in range(2)],
        [pltpu.VMEM((EMBED, CHUNK), jnp.float32) for _ in range(2)],
        [pltpu.VMEM((EMBED, CHUNK), jnp.float32) for _ in range(2)],
        [pltpu.VMEM((EMBED - DTAIL, CHUNK), jnp.float32) for _ in range(2)],
        [pltpu.VMEM((EMBED - DTAIL, CHUNK), jnp.float32) for _ in range(2)],
        pltpu.VMEM((BPW,), jnp.float32),
        pltpu.SemaphoreType.DMA,
    ],
)
def _gmf_sc(c_idx_hbm, u_idx_hbm, c_tab_hbm, u_tab_hbm, ctail_hbm, utail_hbm,
            out_hbm, cidx_v, uidx_v, cib, uib, ctb, utb, cdst, udst,
            ctdst, utdst, out_v, sem):
    wid = lax.axis_index("s") * NUM_CORES + lax.axis_index("c")
    base = wid * BPW

    pltpu.sync_copy(c_idx_hbm.at[pl.ds(base, BPW)], cidx_v)
    pltpu.sync_copy(u_idx_hbm.at[pl.ds(base, BPW)], uidx_v)

    def fire(ch, b):
        for v in range(VPC):
            s = pl.ds(v * LANES, LANES)
            rc = cidx_v[pl.ds(ch * CHUNK + v * LANES, LANES)]
            ru = uidx_v[pl.ds(ch * CHUNK + v * LANES, LANES)]
            tc = ((rc >> 7) << 10) + (rc & 127)
            tu = ((ru >> 7) << 10) + (ru & 127)
            for d in range(EMBED):
                cib[b][d, s] = tc + ((d >> 3) * C_SLAB + (d & 7) * 128)
                uib[b][d, s] = tu + ((d >> 3) * U_SLAB + (d & 7) * 128)
            spread = v * LANES + lax.iota(jnp.int32, LANES)
            rtc = jnp.where(rc >= C_TSTART, rc - C_TSTART, spread)
            rtu = jnp.where(ru >= U_TSTART, ru - U_TSTART, spread)
            for d in range(DTAIL, EMBED):
                ctb[b][d - DTAIL, s] = rtc * EMBED + d
                utb[b][d - DTAIL, s] = rtu * EMBED + d
        copies = []
        for d in range(EMBED):
            copies.append(pltpu.async_copy(
                c_tab_hbm.at[0].at[cib[b].at[d]], cdst[b].at[d], sem))
            copies.append(pltpu.async_copy(
                u_tab_hbm.at[0].at[uib[b].at[d]], udst[b].at[d], sem))
        for j in range(EMBED - DTAIL):
            copies.append(pltpu.async_copy(
                ctail_hbm.at[0].at[ctb[b].at[j]], ctdst[b].at[j], sem))
            copies.append(pltpu.async_copy(
                utail_hbm.at[0].at[utb[b].at[j]], utdst[b].at[j], sem))
        return copies

    def drain(copies):
        for cp in copies:
            cp.wait()

    def compute(ch, b):
        for v in range(VPC):
            s = pl.ds(v * LANES, LANES)
            rc = cidx_v[pl.ds(ch * CHUNK + v * LANES, LANES)]
            ru = uidx_v[pl.ds(ch * CHUNK + v * LANES, LANES)]
            mc = rc >= C_TSTART
            mu = ru >= U_TSTART
            accs = [None] * 4
            for d in range(EMBED):
                cv = cdst[b][d, s]
                uv = udst[b][d, s]
                if d >= DTAIL:
                    cv = jnp.where(mc, ctdst[b][d - DTAIL, s], cv)
                    uv = jnp.where(mu, utdst[b][d - DTAIL, s], uv)
                p = cv * uv
                k = d % 4
                accs[k] = p if accs[k] is None else accs[k] + p
            out_v[pl.ds(ch * CHUNK + v * LANES, LANES)] = (
                (accs[0] + accs[1]) + (accs[2] + accs[3]))

    def pair_body(i, carry):
        ch = i * 2
        ca = fire(ch, 0)
        drain(ca)
        cb = fire(ch + 1, 1)
        compute(ch, 0)
        drain(cb)
        compute(ch + 1, 1)
        return carry

    lax.fori_loop(0, NCH // 2, pair_body, 0)

    pltpu.sync_copy(out_v, out_hbm.at[pl.ds(base, BPW)])


def kernel(c_idx, u_idx, c_table, u_table):
    c_idx32 = jnp.asarray(c_idx, jnp.int32)
    u_idx32 = jnp.asarray(u_idx, jnp.int32)
    fmt = Layout(major_to_minor=(0, 1), tiling=((8, 128),))
    ct = with_layout_constraint(c_table.T, fmt)
    ut = with_layout_constraint(u_table.T, fmt)
    ctail = c_table[C_TSTART:]
    utail = u_table[U_TSTART:]
    out = _gmf_sc(c_idx32, u_idx32, ct, ut, ctail, utail)
    return out.reshape(BATCH, 1)
